# baseline (speedup 1.0000x reference)
import jax
import jax.numpy as jnp
from jax import lax
from jax.experimental import pallas as pl
from jax.experimental.pallas import tpu as pltpu
from jax.experimental.pallas import tpu_sc as plsc

BATCH = 4096
SEQ = 200
DIM = 64
VOCAB = 1000
NC = 2
NS = 16
PG = 4
BG = 8
PP = SEQ // PG
BB = BATCH // BG
NB = 256
NSL = DIM // 16


def _body(xt_hbm, nucleo_hbm, pos_hbm, out_hbm,
          idx_v, table_v, pos_v, obuf0, obuf1, ssem0, ssem1):
    obuf = (obuf0, obuf1)
    ssem = (ssem0, ssem1)
    wid = lax.axis_index("s") * NC + lax.axis_index("c")
    wp = wid % PG
    wb = wid // PG
    p0 = wp * PP
    b0 = wb * BB

    pltpu.sync_copy(nucleo_hbm, table_v)
    pltpu.sync_copy(pos_hbm.at[pl.ds(p0, PP)], pos_v)

    def pstep(p, carry):
        pltpu.sync_copy(xt_hbm.at[p0 + p, pl.ds(b0, BB)], idx_v)
        posr = [pos_v[p, pl.ds(j * 16, 16)] for j in range(NSL)]

        for h in range(2):
            @pl.when(p >= 1)
            def _drain():
                pltpu.make_async_copy(
                    obuf[h],
                    out_hbm.at[pl.ds(b0 + h * NB, NB), p0 + p - 1],
                    ssem[h]).wait()

            def grp(g, carry2):
                xv = idx_v[pl.ds(h * NB + g * 16, 16)]
                for l in range(16):
                    v = xv[l]
                    n = g * 16 + l
                    for j in range(NSL):
                        sl = pl.ds(j * 16, 16)
                        obuf[h][n, sl] = table_v[v, sl] + posr[j]
                return carry2

            lax.fori_loop(0, NB // 16, grp, 0)
            pltpu.async_copy(
                obuf[h], out_hbm.at[pl.ds(b0 + h * NB, NB), p0 + p],
                ssem[h])
        return carry

    lax.fori_loop(0, PP, pstep, 0)

    for h in range(2):
        pltpu.make_async_copy(
            obuf[h], out_hbm.at[pl.ds(b0 + h * NB, NB), p0 + PP - 1],
            ssem[h]).wait()


def kernel(X, nucleo_table, pos_table):
    xt = X.T
    mesh = plsc.VectorSubcoreMesh(core_axis_name="c", subcore_axis_name="s")
    k = pl.kernel(
        _body,
        mesh=mesh,
        compiler_params=pltpu.CompilerParams(use_tc_tiling_on_sc=False),
        out_type=jax.ShapeDtypeStruct((BATCH, SEQ, DIM), jnp.float32),
        scratch_types=[
            pltpu.VMEM((BB,), jnp.int32),
            pltpu.VMEM((VOCAB, DIM), jnp.float32),
            pltpu.VMEM((PP, DIM), jnp.float32),
            pltpu.VMEM((NB, DIM), jnp.float32),
            pltpu.VMEM((NB, DIM), jnp.float32),
            pltpu.SemaphoreType.DMA,
            pltpu.SemaphoreType.DMA,
        ],
    )
    return k(xt, nucleo_table, pos_table)


# speedup vs baseline: 1.2499x; 1.2499x over previous
import jax
import jax.numpy as jnp
from jax import lax
from jax.experimental import pallas as pl
from jax.experimental.pallas import tpu as pltpu
from jax.experimental.pallas import tpu_sc as plsc

BATCH = 4096
SEQ = 200
DIM = 64
VOCAB = 1000
NC = 2
NS = 16
PG = 4
BG = 8
PP = SEQ // PG
BB = BATCH // BG
TB = 16
NT = BB // TB
NSL = DIM // 16


def _body(x_hbm, nucleo_hbm, pos_hbm, out_hbm,
          idx0, idx1, buf0, buf1, pos_v,
          isem0, isem1, gsem0, gsem1, ssem0, ssem1):
    idxv = (idx0, idx1)
    buf = (buf0, buf1)
    isem = (isem0, isem1)
    gsem = (gsem0, gsem1)
    ssem = (ssem0, ssem1)
    wid = lax.axis_index("s") * NC + lax.axis_index("c")
    wp = wid % PG
    wb = wid // PG
    p0 = wp * PP
    b0 = wb * BB

    pltpu.sync_copy(pos_hbm.at[pl.ds(p0, PP)], pos_v)

    def stage_idx(i, s):
        pltpu.async_copy(
            x_hbm.at[wp, pl.ds(b0 + i * TB, TB)], idxv[s], isem[s])

    def fire_gathers(s):
        for n in range(TB):
            pltpu.async_copy(
                nucleo_hbm.at[idxv[s].at[n]], buf[s].at[n], gsem[s])

    def out_slice(i):
        return out_hbm.at[pl.ds(b0 + i * TB, TB), pl.ds(p0, PP)]

    stage_idx(0, 0)
    pltpu.make_async_copy(
        x_hbm.at[wp, pl.ds(b0, TB)], idxv[0], isem[0]).wait()
    fire_gathers(0)

    def pair(g, carry):
        for b in range(2):
            i = g * 2 + b
            s, t = b, 1 - b

            @pl.when(i + 1 < NT)
            def _stage():
                stage_idx(i + 1, t)

                @pl.when(i >= 1)
                def _drain_store():
                    pltpu.make_async_copy(
                        buf[t], out_slice(i - 1), ssem[t]).wait()

                pltpu.make_async_copy(
                    x_hbm.at[wp, pl.ds(b0 + (i + 1) * TB, TB)],
                    idxv[t], isem[t]).wait()
                fire_gathers(t)

            for n in range(TB):
                pltpu.make_async_copy(
                    nucleo_hbm.at[idxv[s].at[n]], buf[s].at[n],
                    gsem[s]).wait()

            def padd(p, carry2):
                posr = [pos_v[p, pl.ds(j * 16, 16)] for j in range(NSL)]
                for n in range(TB):
                    for j in range(NSL):
                        sl = pl.ds(j * 16, 16)
                        buf[s][n, p, sl] = buf[s][n, p, sl] + posr[j]
                return carry2

            lax.fori_loop(0, PP, padd, 0)
            pltpu.async_copy(buf[s], out_slice(i), ssem[s])
        return carry

    lax.fori_loop(0, NT // 2, pair, 0)

    pltpu.make_async_copy(buf[0], out_slice(NT - 2), ssem[0]).wait()
    pltpu.make_async_copy(buf[1], out_slice(NT - 1), ssem[1]).wait()


def kernel(X, nucleo_table, pos_table):
    xb = X.reshape(BATCH, PG, PP).transpose(1, 0, 2)
    mesh = plsc.VectorSubcoreMesh(core_axis_name="c", subcore_axis_name="s")
    k = pl.kernel(
        _body,
        mesh=mesh,
        compiler_params=pltpu.CompilerParams(use_tc_tiling_on_sc=False),
        out_type=jax.ShapeDtypeStruct((BATCH, SEQ, DIM), jnp.float32),
        scratch_types=[
            pltpu.VMEM((TB, PP), jnp.int32),
            pltpu.VMEM((TB, PP), jnp.int32),
            pltpu.VMEM((TB, PP, DIM), jnp.float32),
            pltpu.VMEM((TB, PP, DIM), jnp.float32),
            pltpu.VMEM((PP, DIM), jnp.float32),
            pltpu.SemaphoreType.DMA,
            pltpu.SemaphoreType.DMA,
            pltpu.SemaphoreType.DMA,
            pltpu.SemaphoreType.DMA,
            pltpu.SemaphoreType.DMA,
            pltpu.SemaphoreType.DMA,
        ],
    )
    return k(xb, nucleo_table, pos_table)
